# baseline (speedup 1.0000x reference)
import functools

import jax
import jax.numpy as jnp
from jax import lax
from jax.experimental import pallas as pl
from jax.experimental.pallas import tpu as pltpu
from jax.experimental.pallas import tpu_sc as plsc

Z_COEF = 0.001
BLK = 128
NB = 40
NW = 32


def _router_kernel(x_ref, rw_ref, rb_ref,
                   pos_ref, g_ref, eb_ref, br_ref, z_ref):
    S, E = x_ref.shape[0], rw_ref.shape[1]
    logits = jnp.dot(x_ref[...], rw_ref[...],
                     preferred_element_type=jnp.float32) + rb_ref[...]
    iota = lax.broadcasted_iota(jnp.int32, (S, E), 1).astype(jnp.float32)
    m1 = jnp.max(logits, axis=1, keepdims=True)
    i1 = jnp.min(jnp.where(logits >= m1, iota, float(E)), axis=1,
                 keepdims=True)
    lrest = jnp.where(iota == i1, -jnp.inf, logits)
    m2 = jnp.max(lrest, axis=1, keepdims=True)
    i2 = jnp.min(jnp.where(lrest >= m2, iota, float(E)), axis=1,
                 keepdims=True)
    g1 = 1.0 / (1.0 + jnp.exp(m2 - m1))
    g2 = 1.0 / (1.0 + jnp.exp(m1 - m2))

    lse = jnp.log(jnp.sum(jnp.exp(logits - m1), axis=1, keepdims=True)) + m1
    z_ref[...] = jnp.sum(lse * lse, axis=(0, 1), keepdims=True) * (Z_COEF / S)

    oh1 = (iota == i1).astype(jnp.bfloat16)
    oh2 = (iota == i2).astype(jnp.bfloat16)
    n = (oh1 + oh2)

    ri = lax.broadcasted_iota(jnp.int32, (S, S), 0)
    ci = lax.broadcasted_iota(jnp.int32, (S, S), 1)
    tstrict = (ci < ri).astype(jnp.bfloat16)
    ecum = jnp.dot(tstrict, n, preferred_element_type=jnp.float32)

    rank1 = jnp.sum(ecum * oh1.astype(jnp.float32), axis=1, keepdims=True)
    rank2 = jnp.sum(ecum * oh2.astype(jnp.float32), axis=1, keepdims=True)

    c_row = jnp.sum(n.astype(jnp.float32), axis=0, keepdims=True)
    nb_row = jnp.floor((c_row + float(BLK - 1)) * (1.0 / BLK))
    ue_r = lax.broadcasted_iota(jnp.int32, (E, E), 0)
    ue_c = lax.broadcasted_iota(jnp.int32, (E, E), 1)
    u8 = (ue_r < ue_c).astype(jnp.bfloat16)
    cumnb = jnp.dot(nb_row.astype(jnp.bfloat16), u8,
                    preferred_element_type=jnp.float32)

    start1 = jnp.sum(oh1.astype(jnp.float32) * cumnb, axis=1, keepdims=True)
    start2 = jnp.sum(oh2.astype(jnp.float32) * cumnb, axis=1, keepdims=True)
    pos1 = start1 * float(BLK) + rank1
    pos2 = start2 * float(BLK) + rank2
    pos_ref[...] = jnp.concatenate([pos1, pos2], axis=1).astype(jnp.int32)
    g_ref[...] = jnp.concatenate([g1, g2], axis=1)

    b_iota = lax.broadcasted_iota(jnp.int32, (NB, E), 0).astype(jnp.float32)
    m = (cumnb <= b_iota).astype(jnp.float32)
    eb_ref[...] = (jnp.sum(m, axis=1, keepdims=True) - 1.0).astype(jnp.int32)
    total = jnp.sum(nb_row, axis=1, keepdims=True)
    b0 = lax.broadcasted_iota(jnp.int32, (NB, 1), 0).astype(jnp.float32)
    br_ref[...] = (b0 < total).astype(jnp.int32)


def _sc_dispatch_kernel(x_hbm, t_hbm, p_hbm, xg_hbm, tv, pv, rows_v, sem):
    wid = lax.axis_index("s") * 2 + lax.axis_index("c")
    base = wid * 128
    pltpu.sync_copy(t_hbm.at[pl.ds(base, 128)], tv)
    pltpu.sync_copy(p_hbm.at[pl.ds(base, 128)], pv)
    pltpu.async_copy(x_hbm.at[tv], rows_v, sem).wait()
    pltpu.async_copy(rows_v, xg_hbm.at[pv], sem).wait()


def _sc_combine_kernel(y_hbm, p_hbm, yg_hbm, pv, rows_v, sem):
    wid = lax.axis_index("s") * 2 + lax.axis_index("c")
    for j in range(2):
        base = wid * 128 + j * 64
        pltpu.sync_copy(p_hbm.at[pl.ds(base, 64)], pv)
        pltpu.async_copy(y_hbm.at[pv], rows_v, sem).wait()
        pltpu.sync_copy(rows_v, yg_hbm.at[pl.ds(base, 64)])


def _ffn_kernel(eb_ref, br_ref, xg_ref, wi_ref, wib_ref, wo_ref, wob_ref,
                y_ref):
    b = pl.program_id(0)

    @pl.when(br_ref[b] == 1)
    def _():
        h = jnp.dot(xg_ref[...], wi_ref[0],
                    preferred_element_type=jnp.float32)
        h = jax.nn.gelu(h + wib_ref[0, :, :])
        y = jnp.dot(h.astype(jnp.bfloat16), wo_ref[0],
                    preferred_element_type=jnp.float32)
        y_ref[...] = y + wob_ref[0, :, :]


def _gate_add_kernel(yg_ref, g_ref, o_ref):
    H = o_ref.shape[1]
    o_ref[...] = (yg_ref[:, :H] * g_ref[:, 0:1] +
                  yg_ref[:, H:] * g_ref[:, 1:2])


def kernel(x, router_w, router_b, wi_w, wi_b, wo_w, wo_b):
    B, S, H = x.shape
    E, _, F = wi_w.shape
    A = 2 * S
    P = NB * BLK
    xs = x.reshape(S, H)

    pos, g, eb, br, z = pl.pallas_call(
        _router_kernel,
        out_shape=[
            jax.ShapeDtypeStruct((S, 2), jnp.int32),
            jax.ShapeDtypeStruct((S, 2), jnp.float32),
            jax.ShapeDtypeStruct((NB, 1), jnp.int32),
            jax.ShapeDtypeStruct((NB, 1), jnp.int32),
            jax.ShapeDtypeStruct((1, 1), jnp.float32),
        ],
    )(xs, router_w, router_b.reshape(1, E))

    xbf = xs.astype(jnp.bfloat16)
    xi = lax.bitcast_convert_type(xbf.reshape(S, H // 2, 2), jnp.int32)
    pos_flat = pos.reshape(A)
    t_flat = jnp.arange(A, dtype=jnp.int32) // 2

    mesh = plsc.VectorSubcoreMesh(core_axis_name="c", subcore_axis_name="s")

    xg_i = pl.kernel(
        _sc_dispatch_kernel,
        out_type=jax.ShapeDtypeStruct((P, H // 2), jnp.int32),
        mesh=mesh,
        scratch_types=[
            pltpu.VMEM((128,), jnp.int32),
            pltpu.VMEM((128,), jnp.int32),
            pltpu.VMEM((128, H // 2), jnp.int32),
            pltpu.SemaphoreType.DMA,
        ],
    )(xi, t_flat, pos_flat)
    xg = lax.bitcast_convert_type(xg_i, jnp.bfloat16).reshape(P, H)

    wibf = wi_w.astype(jnp.bfloat16)
    wobf = wo_w.astype(jnp.bfloat16)

    y_all = pl.pallas_call(
        _ffn_kernel,
        grid_spec=pltpu.PrefetchScalarGridSpec(
            num_scalar_prefetch=2,
            grid=(NB,),
            in_specs=[
                pl.BlockSpec((BLK, H), lambda b, eb, br: (b, 0)),
                pl.BlockSpec((1, H, F), lambda b, eb, br: (eb[b], 0, 0)),
                pl.BlockSpec((1, 1, F), lambda b, eb, br: (eb[b], 0, 0)),
                pl.BlockSpec((1, F, H), lambda b, eb, br: (eb[b], 0, 0)),
                pl.BlockSpec((1, 1, H), lambda b, eb, br: (eb[b], 0, 0)),
            ],
            out_specs=pl.BlockSpec((BLK, H), lambda b, eb, br: (b, 0)),
        ),
        out_shape=jax.ShapeDtypeStruct((P, H), jnp.float32),
        compiler_params=pltpu.CompilerParams(
            dimension_semantics=("arbitrary",)),
    )(eb.reshape(NB), br.reshape(NB), xg, wibf,
      wi_b.reshape(E, 1, F), wobf, wo_b.reshape(E, 1, H))

    yg = pl.kernel(
        _sc_combine_kernel,
        out_type=jax.ShapeDtypeStruct((A, H), jnp.float32),
        mesh=mesh,
        scratch_types=[
            pltpu.VMEM((64,), jnp.int32),
            pltpu.VMEM((64, H), jnp.float32),
            pltpu.SemaphoreType.DMA,
        ],
    )(y_all, pos_flat)

    out = pl.pallas_call(
        _gate_add_kernel,
        grid=(4,),
        in_specs=[
            pl.BlockSpec((S // 4, 2 * H), lambda i: (i, 0)),
            pl.BlockSpec((S // 4, 2), lambda i: (i, 0)),
        ],
        out_specs=pl.BlockSpec((S // 4, H), lambda i: (i, 0)),
        out_shape=jax.ShapeDtypeStruct((S, H), jnp.float32),
    )(yg.reshape(S, 2 * H), g)

    return out.reshape(B, S, H), z[0, 0]


# speedup vs baseline: 1.7615x; 1.7615x over previous
import functools

import jax
import jax.numpy as jnp
from jax import lax
from jax.experimental import pallas as pl
from jax.experimental.pallas import tpu as pltpu
from jax.experimental.pallas import tpu_sc as plsc

Z_COEF = 0.001
BLK = 128
NB = 40
NW = 32


def _router_kernel(x_ref, rw_ref, rb_ref, tri_ref,
                   pos_ref, g_ref, eb_ref, br_ref, z_ref):
    S, E = x_ref.shape[0], rw_ref.shape[1]
    logits = jnp.dot(x_ref[...], rw_ref[...],
                     preferred_element_type=jnp.float32) + rb_ref[...]
    iota = lax.broadcasted_iota(jnp.int32, (S, E), 1).astype(jnp.float32)
    m1 = jnp.max(logits, axis=1, keepdims=True)
    i1 = jnp.min(jnp.where(logits >= m1, iota, float(E)), axis=1,
                 keepdims=True)
    lrest = jnp.where(iota == i1, -jnp.inf, logits)
    m2 = jnp.max(lrest, axis=1, keepdims=True)
    i2 = jnp.min(jnp.where(lrest >= m2, iota, float(E)), axis=1,
                 keepdims=True)
    g1 = 1.0 / (1.0 + jnp.exp(m2 - m1))
    g2 = 1.0 / (1.0 + jnp.exp(m1 - m2))

    lse = jnp.log(jnp.sum(jnp.exp(logits - m1), axis=1, keepdims=True)) + m1
    z_ref[...] = jnp.sum(lse * lse, axis=(0, 1), keepdims=True) * (Z_COEF / S)

    oh1 = (iota == i1).astype(jnp.bfloat16)
    oh2 = (iota == i2).astype(jnp.bfloat16)
    n = (oh1 + oh2)

    ecum = jnp.dot(tri_ref[...], n,
                   preferred_element_type=jnp.float32)

    rank1 = jnp.sum(ecum * oh1.astype(jnp.float32), axis=1, keepdims=True)
    rank2 = jnp.sum(ecum * oh2.astype(jnp.float32), axis=1, keepdims=True)

    c_row = jnp.sum(n.astype(jnp.float32), axis=0, keepdims=True)
    nb_row = jnp.floor((c_row + float(BLK - 1)) * (1.0 / BLK))
    ue_r = lax.broadcasted_iota(jnp.int32, (E, E), 0)
    ue_c = lax.broadcasted_iota(jnp.int32, (E, E), 1)
    u8 = (ue_r < ue_c).astype(jnp.bfloat16)
    cumnb = jnp.dot(nb_row.astype(jnp.bfloat16), u8,
                    preferred_element_type=jnp.float32)

    start1 = jnp.sum(oh1.astype(jnp.float32) * cumnb, axis=1, keepdims=True)
    start2 = jnp.sum(oh2.astype(jnp.float32) * cumnb, axis=1, keepdims=True)
    pos1 = start1 * float(BLK) + rank1
    pos2 = start2 * float(BLK) + rank2
    pos_ref[0:1, :] = lax.transpose(pos1, (1, 0)).astype(jnp.int32)
    pos_ref[1:2, :] = lax.transpose(pos2, (1, 0)).astype(jnp.int32)
    g_ref[...] = jnp.concatenate([g1, g2], axis=1)

    b_iota = lax.broadcasted_iota(jnp.int32, (NB, E), 0).astype(jnp.float32)
    m = (cumnb <= b_iota).astype(jnp.float32)
    eb_ref[...] = (jnp.sum(m, axis=1, keepdims=True) - 1.0).astype(jnp.int32)
    total = jnp.sum(nb_row, axis=1, keepdims=True)
    b0 = lax.broadcasted_iota(jnp.int32, (NB, 1), 0).astype(jnp.float32)
    br_ref[...] = (b0 < total).astype(jnp.int32)


def _sc_dispatch_kernel(x_hbm, p_hbm, xg_hbm, pv, rows_v, sem):
    wid = lax.axis_index("s") * 2 + lax.axis_index("c")
    tb = (wid % 16) * 128
    for j in range(2):
        pltpu.sync_copy(x_hbm.at[pl.ds(tb + j * 64, 64)], rows_v)
        pltpu.sync_copy(p_hbm.at[pl.ds(wid * 128 + j * 64, 64)], pv)
        pltpu.async_copy(rows_v, xg_hbm.at[pv], sem).wait()


def _sc_combine_kernel(y_hbm, p_hbm, y0_hbm, y1_hbm, pv, rows_v, sem):
    wid = lax.axis_index("s") * 2 + lax.axis_index("c")
    k = wid // 16
    tb = (wid % 16) * 128
    for j in range(2):
        pltpu.sync_copy(p_hbm.at[pl.ds(wid * 128 + j * 64, 64)], pv)
        pltpu.async_copy(y_hbm.at[pv], rows_v, sem).wait()

        @pl.when(k == 0)
        def _():
            pltpu.sync_copy(rows_v, y0_hbm.at[pl.ds(tb + j * 64, 64)])

        @pl.when(k == 1)
        def _():
            pltpu.sync_copy(rows_v, y1_hbm.at[pl.ds(tb + j * 64, 64)])


def _ffn_kernel(eb_ref, br_ref, xg_ref, wi_ref, wib_ref, wo_ref, wob_ref,
                y_ref):
    b = pl.program_id(0)

    @pl.when(br_ref[b] == 1)
    def _():
        h = jnp.dot(xg_ref[...].astype(jnp.bfloat16), wi_ref[0],
                    preferred_element_type=jnp.float32)
        h = jax.nn.gelu(h + wib_ref[0, :, :])
        y = jnp.dot(h.astype(jnp.bfloat16), wo_ref[0],
                    preferred_element_type=jnp.float32)
        y_ref[...] = y + wob_ref[0, :, :]


def _gate_add_kernel(y0_ref, y1_ref, g_ref, o_ref):
    o_ref[...] = (y0_ref[...] * g_ref[:, 0:1] +
                  y1_ref[...] * g_ref[:, 1:2])


def kernel(x, router_w, router_b, wi_w, wi_b, wo_w, wo_b):
    B, S, H = x.shape
    E, _, F = wi_w.shape
    A = 2 * S
    P = NB * BLK
    xs = x.reshape(S, H)

    tri = jnp.tril(jnp.ones((S, S), jnp.bfloat16), -1)

    pos, g, eb, br, z = pl.pallas_call(
        _router_kernel,
        out_shape=[
            jax.ShapeDtypeStruct((2, S), jnp.int32),
            jax.ShapeDtypeStruct((S, 2), jnp.float32),
            jax.ShapeDtypeStruct((NB, 1), jnp.int32),
            jax.ShapeDtypeStruct((NB, 1), jnp.int32),
            jax.ShapeDtypeStruct((1, 1), jnp.float32),
        ],
    )(xs, router_w, router_b.reshape(1, E), tri)

    pos_flat = pos.reshape(A)

    mesh = plsc.VectorSubcoreMesh(core_axis_name="c", subcore_axis_name="s")

    xg = pl.kernel(
        _sc_dispatch_kernel,
        out_type=jax.ShapeDtypeStruct((P, H), jnp.float32),
        mesh=mesh,
        scratch_types=[
            pltpu.VMEM((64,), jnp.int32),
            pltpu.VMEM((64, H), jnp.float32),
            pltpu.SemaphoreType.DMA,
        ],
    )(xs, pos_flat)

    wibf = wi_w.astype(jnp.bfloat16)
    wobf = wo_w.astype(jnp.bfloat16)

    y_all = pl.pallas_call(
        _ffn_kernel,
        grid_spec=pltpu.PrefetchScalarGridSpec(
            num_scalar_prefetch=2,
            grid=(NB,),
            in_specs=[
                pl.BlockSpec((BLK, H), lambda b, eb, br: (b, 0)),
                pl.BlockSpec((1, H, F), lambda b, eb, br: (eb[b], 0, 0)),
                pl.BlockSpec((1, 1, F), lambda b, eb, br: (eb[b], 0, 0)),
                pl.BlockSpec((1, F, H), lambda b, eb, br: (eb[b], 0, 0)),
                pl.BlockSpec((1, 1, H), lambda b, eb, br: (eb[b], 0, 0)),
            ],
            out_specs=pl.BlockSpec((BLK, H), lambda b, eb, br: (b, 0)),
        ),
        out_shape=jax.ShapeDtypeStruct((P, H), jnp.float32),
        compiler_params=pltpu.CompilerParams(
            dimension_semantics=("arbitrary",)),
    )(eb.reshape(NB), br.reshape(NB), xg, wibf,
      wi_b.reshape(E, 1, F), wobf, wo_b.reshape(E, 1, H))

    y0, y1 = pl.kernel(
        _sc_combine_kernel,
        out_type=[
            jax.ShapeDtypeStruct((S, H), jnp.float32),
            jax.ShapeDtypeStruct((S, H), jnp.float32),
        ],
        mesh=mesh,
        scratch_types=[
            pltpu.VMEM((64,), jnp.int32),
            pltpu.VMEM((64, H), jnp.float32),
            pltpu.SemaphoreType.DMA,
        ],
    )(y_all, pos_flat)

    out = pl.pallas_call(
        _gate_add_kernel,
        grid=(4,),
        in_specs=[
            pl.BlockSpec((S // 4, H), lambda i: (i, 0)),
            pl.BlockSpec((S // 4, H), lambda i: (i, 0)),
            pl.BlockSpec((S // 4, 2), lambda i: (i, 0)),
        ],
        out_specs=pl.BlockSpec((S // 4, H), lambda i: (i, 0)),
        out_shape=jax.ShapeDtypeStruct((S, H), jnp.float32),
    )(y0, y1, g)

    return out.reshape(B, S, H), z[0, 0]


# speedup vs baseline: 2.1783x; 1.2366x over previous
import functools

import jax
import jax.numpy as jnp
from jax import lax
from jax.experimental import pallas as pl
from jax.experimental.pallas import tpu as pltpu
from jax.experimental.pallas import tpu_sc as plsc

Z_COEF = 0.001
BLK = 128
NB = 40
NW = 32


def _router_kernel(x_ref, rw_ref, rb_ref, tri_ref,
                   pos_ref, g_ref, eb_ref, br_ref, z_ref):
    S, E = x_ref.shape[0], rw_ref.shape[1]
    logits = jnp.dot(x_ref[...], rw_ref[...],
                     preferred_element_type=jnp.float32) + rb_ref[...]
    iota = lax.broadcasted_iota(jnp.int32, (S, E), 1).astype(jnp.float32)
    m1 = jnp.max(logits, axis=1, keepdims=True)
    i1 = jnp.min(jnp.where(logits >= m1, iota, float(E)), axis=1,
                 keepdims=True)
    lrest = jnp.where(iota == i1, -jnp.inf, logits)
    m2 = jnp.max(lrest, axis=1, keepdims=True)
    i2 = jnp.min(jnp.where(lrest >= m2, iota, float(E)), axis=1,
                 keepdims=True)
    g1 = 1.0 / (1.0 + jnp.exp(m2 - m1))
    g2 = 1.0 / (1.0 + jnp.exp(m1 - m2))

    lse = jnp.log(jnp.sum(jnp.exp(logits - m1), axis=1, keepdims=True)) + m1
    z_ref[...] = jnp.sum(lse * lse, axis=(0, 1), keepdims=True) * (Z_COEF / S)

    oh1 = (iota == i1).astype(jnp.bfloat16)
    oh2 = (iota == i2).astype(jnp.bfloat16)
    n = (oh1 + oh2)

    ecum = jnp.dot(tri_ref[...], n,
                   preferred_element_type=jnp.float32)

    rank1 = jnp.sum(ecum * oh1.astype(jnp.float32), axis=1, keepdims=True)
    rank2 = jnp.sum(ecum * oh2.astype(jnp.float32), axis=1, keepdims=True)

    c_row = jnp.sum(n.astype(jnp.float32), axis=0, keepdims=True)
    nb_row = jnp.floor((c_row + float(BLK - 1)) * (1.0 / BLK))
    ue_r = lax.broadcasted_iota(jnp.int32, (E, E), 0)
    ue_c = lax.broadcasted_iota(jnp.int32, (E, E), 1)
    u8 = (ue_r < ue_c).astype(jnp.bfloat16)
    cumnb = jnp.dot(nb_row.astype(jnp.bfloat16), u8,
                    preferred_element_type=jnp.float32)

    start1 = jnp.sum(oh1.astype(jnp.float32) * cumnb, axis=1, keepdims=True)
    start2 = jnp.sum(oh2.astype(jnp.float32) * cumnb, axis=1, keepdims=True)
    pos1 = start1 * float(BLK) + rank1
    pos2 = start2 * float(BLK) + rank2
    pos_ref[0:1, :] = lax.transpose(pos1, (1, 0)).astype(jnp.int32)
    pos_ref[1:2, :] = lax.transpose(pos2, (1, 0)).astype(jnp.int32)
    g_ref[...] = jnp.concatenate([g1, g2], axis=1)

    b_iota = lax.broadcasted_iota(jnp.int32, (NB, E), 0).astype(jnp.float32)
    m = (cumnb <= b_iota).astype(jnp.float32)
    eb_ref[...] = (jnp.sum(m, axis=1, keepdims=True) - 1.0).astype(jnp.int32)
    total = jnp.sum(nb_row, axis=1, keepdims=True)
    b0 = lax.broadcasted_iota(jnp.int32, (NB, 1), 0).astype(jnp.float32)
    br_ref[...] = (b0 < total).astype(jnp.int32)


def _sc_dispatch_kernel(x_hbm, p_hbm, xg_hbm, pv, rows_v, sem):
    wid = lax.axis_index("s") * 2 + lax.axis_index("c")
    tb = (wid % 16) * 128
    for j in range(2):
        pltpu.sync_copy(x_hbm.at[pl.ds(tb + j * 64, 64)], rows_v)
        pltpu.sync_copy(p_hbm.at[pl.ds(wid * 128 + j * 64, 64)], pv)
        pltpu.async_copy(rows_v, xg_hbm.at[pv], sem).wait()


def _sc_combine_kernel(y_hbm, p_hbm, y0_hbm, y1_hbm, pv, rows_v, sem):
    wid = lax.axis_index("s") * 2 + lax.axis_index("c")
    k = wid // 16
    tb = (wid % 16) * 128
    for j in range(2):
        pltpu.sync_copy(p_hbm.at[pl.ds(wid * 128 + j * 64, 64)], pv)
        pltpu.async_copy(y_hbm.at[pv], rows_v, sem).wait()

        @pl.when(k == 0)
        def _():
            pltpu.sync_copy(rows_v, y0_hbm.at[pl.ds(tb + j * 64, 64)])

        @pl.when(k == 1)
        def _():
            pltpu.sync_copy(rows_v, y1_hbm.at[pl.ds(tb + j * 64, 64)])


def _ffn_kernel(eb_ref, br_ref, xg_ref, wi_ref, wib_ref, wo_ref, wob_ref,
                y_ref):
    b = pl.program_id(0)

    @pl.when(br_ref[b] == 1)
    def _():
        h = jnp.dot(xg_ref[...], wi_ref[0],
                    preferred_element_type=jnp.float32)
        h = jax.nn.gelu(h + wib_ref[0, :, :])
        y = jnp.dot(h, wo_ref[0], preferred_element_type=jnp.float32)
        y_ref[...] = y + wob_ref[0, :, :]


def _gate_add_kernel(y0_ref, y1_ref, g_ref, o_ref):
    o_ref[...] = (y0_ref[...] * g_ref[:, 0:1] +
                  y1_ref[...] * g_ref[:, 1:2])


def kernel(x, router_w, router_b, wi_w, wi_b, wo_w, wo_b):
    B, S, H = x.shape
    E, _, F = wi_w.shape
    A = 2 * S
    P = NB * BLK
    xs = x.reshape(S, H)

    tri = jnp.tril(jnp.ones((S, S), jnp.bfloat16), -1)

    pos, g, eb, br, z = pl.pallas_call(
        _router_kernel,
        out_shape=[
            jax.ShapeDtypeStruct((2, S), jnp.int32),
            jax.ShapeDtypeStruct((S, 2), jnp.float32),
            jax.ShapeDtypeStruct((NB, 1), jnp.int32),
            jax.ShapeDtypeStruct((NB, 1), jnp.int32),
            jax.ShapeDtypeStruct((1, 1), jnp.float32),
        ],
    )(xs, router_w, router_b.reshape(1, E), tri)

    pos_flat = pos.reshape(A)

    mesh = plsc.VectorSubcoreMesh(core_axis_name="c", subcore_axis_name="s")

    xg = pl.kernel(
        _sc_dispatch_kernel,
        out_type=jax.ShapeDtypeStruct((P, H), jnp.float32),
        mesh=mesh,
        scratch_types=[
            pltpu.VMEM((64,), jnp.int32),
            pltpu.VMEM((64, H), jnp.float32),
            pltpu.SemaphoreType.DMA,
        ],
    )(xs, pos_flat)

    y_all = pl.pallas_call(
        _ffn_kernel,
        grid_spec=pltpu.PrefetchScalarGridSpec(
            num_scalar_prefetch=2,
            grid=(NB,),
            in_specs=[
                pl.BlockSpec((BLK, H), lambda b, eb, br: (b, 0)),
                pl.BlockSpec((1, H, F), lambda b, eb, br: (eb[b], 0, 0)),
                pl.BlockSpec((1, 1, F), lambda b, eb, br: (eb[b], 0, 0)),
                pl.BlockSpec((1, F, H), lambda b, eb, br: (eb[b], 0, 0)),
                pl.BlockSpec((1, 1, H), lambda b, eb, br: (eb[b], 0, 0)),
            ],
            out_specs=pl.BlockSpec((BLK, H), lambda b, eb, br: (b, 0)),
        ),
        out_shape=jax.ShapeDtypeStruct((P, H), jnp.float32),
        compiler_params=pltpu.CompilerParams(
            dimension_semantics=("arbitrary",)),
    )(eb.reshape(NB), br.reshape(NB), xg, wi_w,
      wi_b.reshape(E, 1, F), wo_w, wo_b.reshape(E, 1, H))

    y0, y1 = pl.kernel(
        _sc_combine_kernel,
        out_type=[
            jax.ShapeDtypeStruct((S, H), jnp.float32),
            jax.ShapeDtypeStruct((S, H), jnp.float32),
        ],
        mesh=mesh,
        scratch_types=[
            pltpu.VMEM((64,), jnp.int32),
            pltpu.VMEM((64, H), jnp.float32),
            pltpu.SemaphoreType.DMA,
        ],
    )(y_all, pos_flat)

    out = pl.pallas_call(
        _gate_add_kernel,
        grid=(4,),
        in_specs=[
            pl.BlockSpec((S // 4, H), lambda i: (i, 0)),
            pl.BlockSpec((S // 4, H), lambda i: (i, 0)),
            pl.BlockSpec((S // 4, 2), lambda i: (i, 0)),
        ],
        out_specs=pl.BlockSpec((S // 4, H), lambda i: (i, 0)),
        out_shape=jax.ShapeDtypeStruct((S, H), jnp.float32),
    )(y0, y1, g)

    return out.reshape(B, S, H), z[0, 0]


# speedup vs baseline: 2.2648x; 1.0397x over previous
import functools

import jax
import jax.numpy as jnp
from jax import lax
from jax.experimental import pallas as pl
from jax.experimental.pallas import tpu as pltpu
from jax.experimental.pallas import tpu_sc as plsc

Z_COEF = 0.001
BLK = 256
NB = 24
NW = 32


def _router_kernel(x_ref, rw_ref, rb_ref, tri_ref,
                   pos_ref, g_ref, eb_ref, br_ref, z_ref):
    S, E = x_ref.shape[0], rw_ref.shape[1]
    logits = jnp.dot(x_ref[...], rw_ref[...],
                     preferred_element_type=jnp.float32) + rb_ref[...]
    iota = lax.broadcasted_iota(jnp.int32, (S, E), 1).astype(jnp.float32)
    m1 = jnp.max(logits, axis=1, keepdims=True)
    i1 = jnp.min(jnp.where(logits >= m1, iota, float(E)), axis=1,
                 keepdims=True)
    lrest = jnp.where(iota == i1, -jnp.inf, logits)
    m2 = jnp.max(lrest, axis=1, keepdims=True)
    i2 = jnp.min(jnp.where(lrest >= m2, iota, float(E)), axis=1,
                 keepdims=True)
    g1 = 1.0 / (1.0 + jnp.exp(m2 - m1))
    g2 = 1.0 / (1.0 + jnp.exp(m1 - m2))

    lse = jnp.log(jnp.sum(jnp.exp(logits - m1), axis=1, keepdims=True)) + m1
    z_ref[...] = jnp.sum(lse * lse, axis=(0, 1), keepdims=True) * (Z_COEF / S)

    oh1 = (iota == i1).astype(jnp.bfloat16)
    oh2 = (iota == i2).astype(jnp.bfloat16)
    n = (oh1 + oh2)

    ecum = jnp.dot(tri_ref[...], n,
                   preferred_element_type=jnp.float32)

    rank1 = jnp.sum(ecum * oh1.astype(jnp.float32), axis=1, keepdims=True)
    rank2 = jnp.sum(ecum * oh2.astype(jnp.float32), axis=1, keepdims=True)

    c_row = jnp.sum(n.astype(jnp.float32), axis=0, keepdims=True)
    nb_row = jnp.floor((c_row + float(BLK - 1)) * (1.0 / BLK))
    ue_r = lax.broadcasted_iota(jnp.int32, (E, E), 0)
    ue_c = lax.broadcasted_iota(jnp.int32, (E, E), 1)
    u8 = (ue_r < ue_c).astype(jnp.bfloat16)
    cumnb = jnp.dot(nb_row.astype(jnp.bfloat16), u8,
                    preferred_element_type=jnp.float32)

    start1 = jnp.sum(oh1.astype(jnp.float32) * cumnb, axis=1, keepdims=True)
    start2 = jnp.sum(oh2.astype(jnp.float32) * cumnb, axis=1, keepdims=True)
    pos1 = start1 * float(BLK) + rank1
    pos2 = start2 * float(BLK) + rank2
    pos_ref[0:1, :] = lax.transpose(pos1, (1, 0)).astype(jnp.int32)
    pos_ref[1:2, :] = lax.transpose(pos2, (1, 0)).astype(jnp.int32)
    g_ref[...] = jnp.concatenate([g1, g2], axis=1)

    b_iota = lax.broadcasted_iota(jnp.int32, (NB, E), 0).astype(jnp.float32)
    m = (cumnb <= b_iota).astype(jnp.float32)
    eb_ref[...] = (jnp.sum(m, axis=1, keepdims=True) - 1.0).astype(jnp.int32)
    total = jnp.sum(nb_row, axis=1, keepdims=True)
    b0 = lax.broadcasted_iota(jnp.int32, (NB, 1), 0).astype(jnp.float32)
    br_ref[...] = (b0 < total).astype(jnp.int32)


def _sc_dispatch_kernel(x_hbm, p_hbm, xg_hbm, pv, rows_v, sem):
    wid = lax.axis_index("s") * 2 + lax.axis_index("c")
    tb = (wid % 16) * 128
    for j in range(2):
        pltpu.sync_copy(x_hbm.at[pl.ds(tb + j * 64, 64)], rows_v)
        pltpu.sync_copy(p_hbm.at[pl.ds(wid * 128 + j * 64, 64)], pv)
        pltpu.async_copy(rows_v, xg_hbm.at[pv], sem).wait()


def _sc_combine_kernel(y_hbm, p_hbm, y0_hbm, y1_hbm, pv, rows_v, sem):
    wid = lax.axis_index("s") * 2 + lax.axis_index("c")
    k = wid // 16
    tb = (wid % 16) * 128
    for j in range(2):
        pltpu.sync_copy(p_hbm.at[pl.ds(wid * 128 + j * 64, 64)], pv)
        pltpu.async_copy(y_hbm.at[pv], rows_v, sem).wait()

        @pl.when(k == 0)
        def _():
            pltpu.sync_copy(rows_v, y0_hbm.at[pl.ds(tb + j * 64, 64)])

        @pl.when(k == 1)
        def _():
            pltpu.sync_copy(rows_v, y1_hbm.at[pl.ds(tb + j * 64, 64)])


def _ffn_kernel(eb_ref, br_ref, xg_ref, wi_ref, wib_ref, wo_ref, wob_ref,
                y_ref):
    b = pl.program_id(0)

    @pl.when(br_ref[b] == 1)
    def _():
        h = jnp.dot(xg_ref[...], wi_ref[0],
                    preferred_element_type=jnp.float32)
        h = jax.nn.gelu(h + wib_ref[0, :, :])
        y = jnp.dot(h, wo_ref[0], preferred_element_type=jnp.float32)
        y_ref[...] = y + wob_ref[0, :, :]


def _gate_add_kernel(y0_ref, y1_ref, g_ref, o_ref):
    o_ref[...] = (y0_ref[...] * g_ref[:, 0:1] +
                  y1_ref[...] * g_ref[:, 1:2])


def kernel(x, router_w, router_b, wi_w, wi_b, wo_w, wo_b):
    B, S, H = x.shape
    E, _, F = wi_w.shape
    A = 2 * S
    P = NB * BLK
    xs = x.reshape(S, H)

    tri = jnp.tril(jnp.ones((S, S), jnp.bfloat16), -1)

    pos, g, eb, br, z = pl.pallas_call(
        _router_kernel,
        out_shape=[
            jax.ShapeDtypeStruct((2, S), jnp.int32),
            jax.ShapeDtypeStruct((S, 2), jnp.float32),
            jax.ShapeDtypeStruct((NB, 1), jnp.int32),
            jax.ShapeDtypeStruct((NB, 1), jnp.int32),
            jax.ShapeDtypeStruct((1, 1), jnp.float32),
        ],
    )(xs, router_w, router_b.reshape(1, E), tri)

    pos_flat = pos.reshape(A)

    mesh = plsc.VectorSubcoreMesh(core_axis_name="c", subcore_axis_name="s")

    xg = pl.kernel(
        _sc_dispatch_kernel,
        out_type=jax.ShapeDtypeStruct((P, H), jnp.float32),
        mesh=mesh,
        scratch_types=[
            pltpu.VMEM((64,), jnp.int32),
            pltpu.VMEM((64, H), jnp.float32),
            pltpu.SemaphoreType.DMA,
        ],
    )(xs, pos_flat)

    y_all = pl.pallas_call(
        _ffn_kernel,
        grid_spec=pltpu.PrefetchScalarGridSpec(
            num_scalar_prefetch=2,
            grid=(NB,),
            in_specs=[
                pl.BlockSpec((BLK, H), lambda b, eb, br: (b, 0)),
                pl.BlockSpec((1, H, F), lambda b, eb, br: (eb[b], 0, 0)),
                pl.BlockSpec((1, 1, F), lambda b, eb, br: (eb[b], 0, 0)),
                pl.BlockSpec((1, F, H), lambda b, eb, br: (eb[b], 0, 0)),
                pl.BlockSpec((1, 1, H), lambda b, eb, br: (eb[b], 0, 0)),
            ],
            out_specs=pl.BlockSpec((BLK, H), lambda b, eb, br: (b, 0)),
        ),
        out_shape=jax.ShapeDtypeStruct((P, H), jnp.float32),
        compiler_params=pltpu.CompilerParams(
            dimension_semantics=("arbitrary",)),
    )(eb.reshape(NB), br.reshape(NB), xg, wi_w,
      wi_b.reshape(E, 1, F), wo_w, wo_b.reshape(E, 1, H))

    y0, y1 = pl.kernel(
        _sc_combine_kernel,
        out_type=[
            jax.ShapeDtypeStruct((S, H), jnp.float32),
            jax.ShapeDtypeStruct((S, H), jnp.float32),
        ],
        mesh=mesh,
        scratch_types=[
            pltpu.VMEM((64,), jnp.int32),
            pltpu.VMEM((64, H), jnp.float32),
            pltpu.SemaphoreType.DMA,
        ],
    )(y_all, pos_flat)

    out = pl.pallas_call(
        _gate_add_kernel,
        grid=(4,),
        in_specs=[
            pl.BlockSpec((S // 4, H), lambda i: (i, 0)),
            pl.BlockSpec((S // 4, H), lambda i: (i, 0)),
            pl.BlockSpec((S // 4, 2), lambda i: (i, 0)),
        ],
        out_specs=pl.BlockSpec((S // 4, H), lambda i: (i, 0)),
        out_shape=jax.ShapeDtypeStruct((S, H), jnp.float32),
    )(y0, y1, g)

    return out.reshape(B, S, H), z[0, 0]
